# baseline (speedup 1.0000x reference)
import functools

import jax
import jax.numpy as jnp
from jax import lax
from jax.experimental import pallas as pl
from jax.experimental.pallas import tpu as pltpu
from jax.experimental.pallas import tpu_sc as plsc

N = 320000
D = 128
G = 1024
EPS = 1e-5
NEG = -3.0e38

NC = 2
NS = 16
NW = NC * NS
RPW = N // NW
CHUNK = 400
NCHUNK = RPW // CHUNK
NRING = 2
NB32 = CHUNK // 32
NBLK = CHUNK // 16
NL = D // 16


def _scalar_store(ref, idx, val):
  idx = jnp.asarray(idx, jnp.int32)
  base = (idx // 16) * 16
  lane = idx - base
  lanes = lax.broadcasted_iota(jnp.int32, (16,), 0)
  cur = ref[pl.ds(base, 16)]
  ref[pl.ds(base, 16)] = jnp.where(lanes == lane, jnp.full((16,), val), cur)


def _sc_reduce_build():
  mesh = plsc.VectorSubcoreMesh(core_axis_name="c", subcore_axis_name="s")
  out_type = (
      jax.ShapeDtypeStruct((G, 2 * D), jnp.float32),
      jax.ShapeDtypeStruct((NW, G), jnp.float32),
      jax.ShapeDtypeStruct((NW, 16), jnp.int32),
      jax.ShapeDtypeStruct((NW, 16), jnp.float32),
      jax.ShapeDtypeStruct((NW, 4 * D), jnp.float32),
  )
  scratch = [
      pltpu.VMEM((CHUNK, D), jnp.float32),
      pltpu.VMEM((CHUNK, D), jnp.float32),
      pltpu.VMEM((RPW + 16,), jnp.int32),
      pltpu.VMEM((2, 2 * D), jnp.float32),
      pltpu.VMEM((G,), jnp.float32),
      pltpu.VMEM((4 * D,), jnp.float32),
      pltpu.VMEM((16,), jnp.float32),
      pltpu.VMEM((16,), jnp.int32),
      pltpu.SemaphoreType.DMA,
      pltpu.SemaphoreType.DMA,
      pltpu.SemaphoreType.DMA,
      pltpu.SemaphoreType.DMA,
  ]

  @functools.partial(pl.kernel, out_type=out_type, mesh=mesh,
                     scratch_types=scratch)
  def sc_reduce(ne, bt, ism, icnt, esid, ecnt, esm,
                rb0, rb1, bb, acc, cloc, esl, ec, eid,
                smr0, smr1, smb, smf):
    cid = lax.axis_index("c")
    sid = lax.axis_index("s")
    wid = sid * NC + cid
    row0 = wid * RPW

    def start_rows(c, rb, smr):
      pltpu.make_async_copy(ne.at[pl.ds(row0 + c * CHUNK, CHUNK)], rb,
                            smr).start()

    def wait_rows(rb, smr):
      pltpu.make_async_copy(ne.at[pl.ds(0, CHUNK)], rb, smr).wait()

    def wait_flush():
      pltpu.make_async_copy(acc.at[0], ism.at[0], smf).wait()

    rbufs = (rb0, rb1)
    rsems = (smr0, smr1)

    pltpu.make_async_copy(bt.at[pl.ds(row0, RPW)],
                          bb.at[pl.ds(0, RPW)], smb).start()
    for b in range(NRING):
      start_rows(b, rbufs[b], rsems[b])

    def zcnt(i, _):
      cloc[pl.ds(i * 16, 16)] = jnp.zeros((16,), jnp.float32)
      return 0
    lax.fori_loop(0, G // 16, zcnt, 0)
    eid[...] = jnp.full((16,), -1, jnp.int32)
    ec[...] = jnp.zeros((16,), jnp.float32)

    pltpu.make_async_copy(bt.at[pl.ds(0, RPW)],
                          bb.at[pl.ds(0, RPW)], smb).wait()

    def save_edge(slot, seg, cnt, nfl):
      p = nfl & 1
      for j in range(NL):
        esl[pl.ds(slot * 2 * D + j * 16, 16)] = acc[p, pl.ds(j * 16, 16)]
        esl[pl.ds(slot * 2 * D + D + j * 16, 16)] = (
            acc[p, pl.ds(D + j * 16, 16)])
      _scalar_store(ec, slot, cnt.astype(jnp.float32))
      _scalar_store(eid, slot, seg)

    def make_row_body(rb):
      def row_body(args):
        i, ci, carry = args
        seg, cnt, nfl = carry
        b = bb[pl.ds(i, 16)][0]
        boundary = b != seg
        do_flush = jnp.logical_and(boundary, cnt > 0)

        @pl.when(do_flush)
        def _():
          p = nfl & 1

          @pl.when(nfl == 0)
          def _():
            save_edge(0, seg, cnt, nfl)

          @pl.when(nfl > 0)
          def _():
            @pl.when(nfl > 1)
            def _():
              wait_flush()
            pltpu.make_async_copy(acc.at[p], ism.at[seg], smf).start()
            _scalar_store(cloc, seg, cnt.astype(jnp.float32))

        newseg = jnp.where(boundary, b, seg)
        newcnt = jnp.where(boundary, 1, cnt + 1)
        newnfl = jnp.where(do_flush, nfl + 1, nfl)
        q = newnfl & 1
        for j in range(NL):
          v = rb[ci, pl.ds(j * 16, 16)]
          os_ = acc[q, pl.ds(j * 16, 16)]
          om_ = acc[q, pl.ds(D + j * 16, 16)]
          acc[q, pl.ds(j * 16, 16)] = jnp.where(boundary, v, os_ + v)
          acc[q, pl.ds(D + j * 16, 16)] = jnp.where(
              boundary, v, jnp.maximum(om_, v))
        return (newseg, newcnt, newnfl)
      return row_body

    def make_chunk_proc(rb):
      row_body = make_row_body(rb)

      def make_block(cbase, koff, W):
        def block_body(k, cr):
          seg = cr[0]
          base = koff + k * W
          first = bb[pl.ds(cbase + base, 16)][0]
          last = bb[pl.ds(cbase + base + W - 16, 16)][15]
          fast = jnp.logical_and(first == seg, last == seg)

          def fast_fn(c2):
            p = c2[2] & 1
            s_ = [acc[p, pl.ds(j * 16, 16)] for j in range(NL)]
            m_ = [acc[p, pl.ds(D + j * 16, 16)] for j in range(NL)]
            for r in range(W):
              for j in range(NL):
                v = rb[base + r, pl.ds(j * 16, 16)]
                s_[j] = s_[j] + v
                m_[j] = jnp.maximum(m_[j], v)
            for j in range(NL):
              acc[p, pl.ds(j * 16, 16)] = s_[j]
              acc[p, pl.ds(D + j * 16, 16)] = m_[j]
            return (c2[0], c2[1] + W, c2[2])

          def slow_fn(c2):
            def rb_body(r, c3):
              return row_body((cbase + base + r, base + r, c3))
            return lax.fori_loop(0, W, rb_body, c2)

          return lax.cond(fast, fast_fn, slow_fn, cr)
        return block_body

      def proc(carry, cbase):
        carry = lax.fori_loop(0, NB32, make_block(cbase, 0, 32), carry)
        return make_block(cbase, NB32 * 32, 16)(0, carry)
      return proc

    procs = [make_chunk_proc(rb) for rb in rbufs]

    def ring_body(i, carry):
      for b in range(NRING):
        c = NRING * i + b
        wait_rows(rbufs[b], rsems[b])
        carry = procs[b](carry, c * CHUNK)

        @pl.when(c + NRING < NCHUNK)
        def _():
          start_rows(c + NRING, rbufs[b], rsems[b])
      return carry

    init = (jnp.int32(-1), jnp.int32(0), jnp.int32(0))
    carry = lax.fori_loop(0, NCHUNK // NRING, ring_body, init)
    for b in range(NCHUNK % NRING):
      c = (NCHUNK // NRING) * NRING + b
      wait_rows(rbufs[b], rsems[b])
      carry = procs[b](carry, c * CHUNK)

    seg, cnt, nfl = carry

    @pl.when(nfl == 0)
    def _():
      save_edge(0, seg, cnt, nfl)

    @pl.when(nfl > 0)
    def _():
      save_edge(1, seg, cnt, nfl)

    @pl.when(nfl >= 2)
    def _():
      wait_flush()

    pltpu.sync_copy(cloc, icnt.at[wid])
    pltpu.sync_copy(eid, esid.at[wid])
    pltpu.sync_copy(ec, ecnt.at[wid])
    pltpu.sync_copy(esl, esm.at[wid])

  return sc_reduce


_sc_reduce = _sc_reduce_build()


def _tc_head(ism, icnt, esid, esidv, ecntv, esm,
             ln_g, ln_b, W1, b1, W2, b2, Ws, bs, Wc, bc,
             score_out, cls_out, mx):
  dn = (((0,), (1,)), ((), ()))
  ones_w = jnp.ones((1, NW), jnp.float32)
  cnt_int = lax.dot_general(icnt[...], ones_w, dn,
                            preferred_element_type=jnp.float32)
  valid = cnt_int > 0.0

  isum = ism[:, pl.ds(0, D)]
  imax = ism[:, pl.ds(D, D)]

  mx[pl.ds(0, G), :] = jnp.where(valid, imax, NEG)
  mx[pl.ds(G, 8), :] = jnp.full((8, D), NEG, jnp.float32)

  def merge_max(w, _):
    for slot in range(2):
      sd = esid[w, slot]
      tid = jnp.where(sd >= 0, sd, G)
      rowf = esm[pl.ds(w, 1), :]
      row = lax.slice(rowf, (0, (2 * slot + 1) * D), (1, (2 * slot + 2) * D))
      cur = mx[pl.ds(tid, 1), :]
      mx[pl.ds(tid, 1), :] = jnp.maximum(cur, row)
    return 0
  lax.fori_loop(0, NW, merge_max, 0)

  ids = jax.lax.broadcasted_iota(jnp.int32, (NW, G), 1)
  oh0 = (esidv[:, pl.ds(0, 1)] == ids).astype(jnp.float32)
  oh1 = (esidv[:, pl.ds(1, 1)] == ids).astype(jnp.float32)
  dn0 = (((0,), (0,)), ((), ()))
  s = jnp.where(valid, isum, 0.0)
  s = s + lax.dot_general(oh0, esm[:, pl.ds(0, D)], dn0,
                          preferred_element_type=jnp.float32)
  s = s + lax.dot_general(oh1, esm[:, pl.ds(2 * D, D)], dn0,
                          preferred_element_type=jnp.float32)
  cnt = cnt_int
  cnt = cnt + lax.dot_general(oh0, ecntv[:, pl.ds(0, 1)], dn0,
                              preferred_element_type=jnp.float32)
  cnt = cnt + lax.dot_general(oh1, ecntv[:, pl.ds(1, 1)], dn0,
                              preferred_element_type=jnp.float32)

  mean = s / jnp.maximum(cnt, 1.0)
  mfin = jnp.where(cnt > 0.0, mx[pl.ds(0, G), :], 0.0)

  g = jnp.concatenate([s, mean, mfin], axis=1)
  mu = jnp.mean(g, axis=1, keepdims=True)
  var = jnp.mean((g - mu) ** 2, axis=1, keepdims=True)
  h = (g - mu) * jax.lax.rsqrt(var + EPS) * ln_g[...] + ln_b[...]

  h = h @ W1[...] + b1[...]
  h = h * jax.nn.sigmoid(h)
  h = h @ W2[...] + b2[...]
  h = h * jax.nn.sigmoid(h)
  score_out[...] = jax.nn.sigmoid(h @ Ws[...] + bs[...])
  cls_out[...] = h @ Wc[...] + bc[...]


@jax.jit
def kernel(node_emb, batch, ln_g, ln_b, W1, b1, W2, b2, Ws, bs, Wc, bc):
  ism, icnt, esid, ecnt, esm = _sc_reduce(node_emb, batch)

  vspec = pl.BlockSpec(memory_space=pltpu.VMEM)
  sspec = pl.BlockSpec(memory_space=pltpu.SMEM)
  score, cls = pl.pallas_call(
      _tc_head,
      out_shape=[jax.ShapeDtypeStruct((G, 1), jnp.float32),
                 jax.ShapeDtypeStruct((G, 4), jnp.float32)],
      in_specs=[vspec, vspec, sspec] + [vspec] * 13,
      out_specs=[vspec, vspec],
      scratch_shapes=[pltpu.VMEM((G + 8, D), jnp.float32)],
  )(ism, icnt, esid, esid, ecnt, esm,
    ln_g.reshape(1, 3 * D), ln_b.reshape(1, 3 * D), W1, b1.reshape(1, D),
    W2, b2.reshape(1, D // 2), Ws, bs.reshape(1, 1), Wc, bc.reshape(1, 4))
  return score[:, 0], cls


# speedup vs baseline: 1.3471x; 1.3471x over previous
import functools

import jax
import jax.numpy as jnp
from jax import lax
from jax.experimental import pallas as pl
from jax.experimental.pallas import tpu as pltpu
from jax.experimental.pallas import tpu_sc as plsc

N = 320000
D = 128
G = 1024
EPS = 1e-5
NEG = -3.0e38

NC = 2
NS = 16
NW = NC * NS
RPW = N // NW
CHUNK = 400
NCHUNK = RPW // CHUNK
NRING = 2
NBLK = CHUNK // 16
NBLK = CHUNK // 16
NL = D // 16


def _scalar_store(ref, idx, val):
  idx = jnp.asarray(idx, jnp.int32)
  base = (idx // 16) * 16
  lane = idx - base
  lanes = lax.broadcasted_iota(jnp.int32, (16,), 0)
  cur = ref[pl.ds(base, 16)]
  ref[pl.ds(base, 16)] = jnp.where(lanes == lane, jnp.full((16,), val), cur)


def _sc_reduce_build():
  mesh = plsc.VectorSubcoreMesh(core_axis_name="c", subcore_axis_name="s")
  out_type = (
      jax.ShapeDtypeStruct((G, 2 * D), jnp.float32),
      jax.ShapeDtypeStruct((NW, G), jnp.float32),
      jax.ShapeDtypeStruct((NW, 16), jnp.int32),
      jax.ShapeDtypeStruct((NW, 16), jnp.float32),
      jax.ShapeDtypeStruct((NW, 4 * D), jnp.float32),
  )
  scratch = [
      pltpu.VMEM((CHUNK, D), jnp.float32),
      pltpu.VMEM((CHUNK, D), jnp.float32),
      pltpu.VMEM((RPW + 16,), jnp.int32),
      pltpu.VMEM((2, 2 * D), jnp.float32),
      pltpu.VMEM((G,), jnp.float32),
      pltpu.VMEM((4 * D,), jnp.float32),
      pltpu.VMEM((16,), jnp.float32),
      pltpu.VMEM((16,), jnp.int32),
      pltpu.SemaphoreType.DMA,
      pltpu.SemaphoreType.DMA,
      pltpu.SemaphoreType.DMA,
      pltpu.SemaphoreType.DMA,
  ]

  @functools.partial(pl.kernel, out_type=out_type, mesh=mesh,
                     scratch_types=scratch)
  def sc_reduce(ne, bt, ism, icnt, esid, ecnt, esm,
                rb0, rb1, bb, acc, cloc, esl, ec, eid,
                smr0, smr1, smb, smf):
    cid = lax.axis_index("c")
    sid = lax.axis_index("s")
    wid = sid * NC + cid
    row0 = wid * RPW

    def start_rows(c, rb, smr):
      pltpu.make_async_copy(ne.at[pl.ds(row0 + c * CHUNK, CHUNK)], rb,
                            smr).start()

    def wait_rows(rb, smr):
      pltpu.make_async_copy(ne.at[pl.ds(0, CHUNK)], rb, smr).wait()

    def wait_flush():
      pltpu.make_async_copy(acc.at[0], ism.at[0], smf).wait()

    rbufs = (rb0, rb1)
    rsems = (smr0, smr1)

    pltpu.make_async_copy(bt.at[pl.ds(row0, RPW)],
                          bb.at[pl.ds(0, RPW)], smb).start()
    for b in range(NRING):
      start_rows(b, rbufs[b], rsems[b])

    def zcnt(i, _):
      cloc[pl.ds(i * 16, 16)] = jnp.zeros((16,), jnp.float32)
      return 0
    lax.fori_loop(0, G // 16, zcnt, 0)
    eid[...] = jnp.full((16,), -1, jnp.int32)
    ec[...] = jnp.zeros((16,), jnp.float32)

    pltpu.make_async_copy(bt.at[pl.ds(0, RPW)],
                          bb.at[pl.ds(0, RPW)], smb).wait()

    def save_edge(slot, seg, cnt, nfl):
      p = nfl & 1
      for j in range(NL):
        esl[pl.ds(slot * 2 * D + j * 16, 16)] = acc[p, pl.ds(j * 16, 16)]
        esl[pl.ds(slot * 2 * D + D + j * 16, 16)] = (
            acc[p, pl.ds(D + j * 16, 16)])
      _scalar_store(ec, slot, cnt.astype(jnp.float32))
      _scalar_store(eid, slot, seg)

    def make_row_body(rb):
      def row_body(args):
        i, ci, carry = args
        seg, cnt, nfl = carry
        b = bb[pl.ds(i, 16)][0]
        boundary = b != seg
        do_flush = jnp.logical_and(boundary, cnt > 0)

        @pl.when(do_flush)
        def _():
          p = nfl & 1

          @pl.when(nfl == 0)
          def _():
            save_edge(0, seg, cnt, nfl)

          @pl.when(nfl > 0)
          def _():
            @pl.when(nfl > 1)
            def _():
              wait_flush()
            pltpu.make_async_copy(acc.at[p], ism.at[seg], smf).start()
            _scalar_store(cloc, seg, cnt.astype(jnp.float32))

        newseg = jnp.where(boundary, b, seg)
        newcnt = jnp.where(boundary, 1, cnt + 1)
        newnfl = jnp.where(do_flush, nfl + 1, nfl)
        q = newnfl & 1
        for j in range(NL):
          v = rb[ci, pl.ds(j * 16, 16)]
          os_ = acc[q, pl.ds(j * 16, 16)]
          om_ = acc[q, pl.ds(D + j * 16, 16)]
          acc[q, pl.ds(j * 16, 16)] = jnp.where(boundary, v, os_ + v)
          acc[q, pl.ds(D + j * 16, 16)] = jnp.where(
              boundary, v, jnp.maximum(om_, v))
        return (newseg, newcnt, newnfl)
      return row_body

    def make_chunk_proc(rb):
      row_body = make_row_body(rb)

      def make_block(cbase, koff, W):
        def block_body(k, cr):
          seg = cr[0]
          base = koff + k * W
          first = bb[pl.ds(cbase + base, 16)][0]
          last = bb[pl.ds(cbase + base + W - 16, 16)][15]
          fast = jnp.logical_and(first == seg, last == seg)

          def fast_fn(c2):
            p = c2[2] & 1
            s_ = [acc[p, pl.ds(j * 16, 16)] for j in range(NL)]
            m_ = [acc[p, pl.ds(D + j * 16, 16)] for j in range(NL)]
            for r in range(W):
              for j in range(NL):
                v = rb[base + r, pl.ds(j * 16, 16)]
                s_[j] = s_[j] + v
                m_[j] = jnp.maximum(m_[j], v)
            for j in range(NL):
              acc[p, pl.ds(j * 16, 16)] = s_[j]
              acc[p, pl.ds(D + j * 16, 16)] = m_[j]
            return (c2[0], c2[1] + W, c2[2])

          def slow_fn(c2):
            def rb_body(r, c3):
              return row_body((cbase + base + r, base + r, c3))
            return lax.fori_loop(0, W, rb_body, c2)

          return lax.cond(fast, fast_fn, slow_fn, cr)
        return block_body

      def proc(carry, cbase):
        return lax.fori_loop(0, NBLK, make_block(cbase, 0, 16), carry)
      return proc

    procs = [make_chunk_proc(rb) for rb in rbufs]

    def ring_body(i, carry):
      for b in range(NRING):
        c = NRING * i + b
        wait_rows(rbufs[b], rsems[b])
        carry = procs[b](carry, c * CHUNK)

        @pl.when(c + NRING < NCHUNK)
        def _():
          start_rows(c + NRING, rbufs[b], rsems[b])
      return carry

    init = (jnp.int32(-1), jnp.int32(0), jnp.int32(0))
    carry = lax.fori_loop(0, NCHUNK // NRING, ring_body, init)
    for b in range(NCHUNK % NRING):
      c = (NCHUNK // NRING) * NRING + b
      wait_rows(rbufs[b], rsems[b])
      carry = procs[b](carry, c * CHUNK)

    seg, cnt, nfl = carry

    @pl.when(nfl == 0)
    def _():
      save_edge(0, seg, cnt, nfl)

    @pl.when(nfl > 0)
    def _():
      save_edge(1, seg, cnt, nfl)

    @pl.when(nfl >= 2)
    def _():
      wait_flush()

    pltpu.sync_copy(cloc, icnt.at[wid])
    pltpu.sync_copy(eid, esid.at[wid])
    pltpu.sync_copy(ec, ecnt.at[wid])
    pltpu.sync_copy(esl, esm.at[wid])

  return sc_reduce


_sc_reduce = _sc_reduce_build()


def _tc_head(ism, icnt, esid, esidv, ecntv, esm,
             ln_g, ln_b, W1, b1, W2, b2, Ws, bs, Wc, bc,
             score_out, cls_out, mx):
  dn = (((0,), (1,)), ((), ()))
  ones_w = jnp.ones((1, NW), jnp.float32)
  cnt_int = lax.dot_general(icnt[...], ones_w, dn,
                            preferred_element_type=jnp.float32)
  valid = cnt_int > 0.0

  isum = ism[:, pl.ds(0, D)]
  imax = ism[:, pl.ds(D, D)]

  mx[pl.ds(0, G), :] = jnp.where(valid, imax, NEG)
  mx[pl.ds(G, 8), :] = jnp.full((8, D), NEG, jnp.float32)

  def merge_max(w, _):
    for slot in range(2):
      sd = esid[w, slot]
      tid = jnp.where(sd >= 0, sd, G)
      rowf = esm[pl.ds(w, 1), :]
      row = lax.slice(rowf, (0, (2 * slot + 1) * D), (1, (2 * slot + 2) * D))
      cur = mx[pl.ds(tid, 1), :]
      mx[pl.ds(tid, 1), :] = jnp.maximum(cur, row)
    return 0
  lax.fori_loop(0, NW, merge_max, 0)

  ids = jax.lax.broadcasted_iota(jnp.int32, (NW, G), 1)
  oh0 = (esidv[:, pl.ds(0, 1)] == ids).astype(jnp.float32)
  oh1 = (esidv[:, pl.ds(1, 1)] == ids).astype(jnp.float32)
  dn0 = (((0,), (0,)), ((), ()))
  s = jnp.where(valid, isum, 0.0)
  s = s + lax.dot_general(oh0, esm[:, pl.ds(0, D)], dn0,
                          preferred_element_type=jnp.float32)
  s = s + lax.dot_general(oh1, esm[:, pl.ds(2 * D, D)], dn0,
                          preferred_element_type=jnp.float32)
  cnt = cnt_int
  cnt = cnt + lax.dot_general(oh0, ecntv[:, pl.ds(0, 1)], dn0,
                              preferred_element_type=jnp.float32)
  cnt = cnt + lax.dot_general(oh1, ecntv[:, pl.ds(1, 1)], dn0,
                              preferred_element_type=jnp.float32)

  mean = s / jnp.maximum(cnt, 1.0)
  mfin = jnp.where(cnt > 0.0, mx[pl.ds(0, G), :], 0.0)

  g = jnp.concatenate([s, mean, mfin], axis=1)
  mu = jnp.mean(g, axis=1, keepdims=True)
  var = jnp.mean((g - mu) ** 2, axis=1, keepdims=True)
  h = (g - mu) * jax.lax.rsqrt(var + EPS) * ln_g[...] + ln_b[...]

  h = h @ W1[...] + b1[...]
  h = h * jax.nn.sigmoid(h)
  h = h @ W2[...] + b2[...]
  h = h * jax.nn.sigmoid(h)
  score_out[...] = jax.nn.sigmoid(h @ Ws[...] + bs[...])
  cls_out[...] = h @ Wc[...] + bc[...]


@jax.jit
def kernel(node_emb, batch, ln_g, ln_b, W1, b1, W2, b2, Ws, bs, Wc, bc):
  ism, icnt, esid, ecnt, esm = _sc_reduce(node_emb, batch)

  vspec = pl.BlockSpec(memory_space=pltpu.VMEM)
  sspec = pl.BlockSpec(memory_space=pltpu.SMEM)
  score, cls = pl.pallas_call(
      _tc_head,
      out_shape=[jax.ShapeDtypeStruct((G, 1), jnp.float32),
                 jax.ShapeDtypeStruct((G, 4), jnp.float32)],
      in_specs=[vspec, vspec, sspec] + [vspec] * 13,
      out_specs=[vspec, vspec],
      scratch_shapes=[pltpu.VMEM((G + 8, D), jnp.float32)],
  )(ism, icnt, esid, esid, ecnt, esm,
    ln_g.reshape(1, 3 * D), ln_b.reshape(1, 3 * D), W1, b1.reshape(1, D),
    W2, b2.reshape(1, D // 2), Ws, bs.reshape(1, 1), Wc, bc.reshape(1, 4))
  return score[:, 0], cls


# speedup vs baseline: 1.3623x; 1.0113x over previous
import functools

import jax
import jax.numpy as jnp
from jax import lax
from jax.experimental import pallas as pl
from jax.experimental.pallas import tpu as pltpu
from jax.experimental.pallas import tpu_sc as plsc

N = 320000
D = 128
G = 1024
EPS = 1e-5
NEG = -3.0e38

NC = 2
NS = 16
NW = NC * NS
RPW = N // NW
CHUNK = 400
NCHUNK = RPW // CHUNK
NRING = 2
NBLK = CHUNK // 16
NBLK = CHUNK // 16
NL = D // 16


def _scalar_store(ref, idx, val):
  idx = jnp.asarray(idx, jnp.int32)
  base = (idx // 16) * 16
  lane = idx - base
  lanes = lax.broadcasted_iota(jnp.int32, (16,), 0)
  cur = ref[pl.ds(base, 16)]
  ref[pl.ds(base, 16)] = jnp.where(lanes == lane, jnp.full((16,), val), cur)


def _sc_reduce_build():
  mesh = plsc.VectorSubcoreMesh(core_axis_name="c", subcore_axis_name="s")
  out_type = (
      jax.ShapeDtypeStruct((G, 2 * D), jnp.float32),
      jax.ShapeDtypeStruct((NW, G), jnp.float32),
      jax.ShapeDtypeStruct((NW, 16), jnp.int32),
      jax.ShapeDtypeStruct((NW, 16), jnp.float32),
      jax.ShapeDtypeStruct((NW, 4 * D), jnp.float32),
  )
  scratch = [
      pltpu.VMEM((CHUNK, D), jnp.float32),
      pltpu.VMEM((CHUNK, D), jnp.float32),
      pltpu.VMEM((RPW + 16,), jnp.int32),
      pltpu.VMEM((2, 2 * D), jnp.float32),
      pltpu.VMEM((G,), jnp.float32),
      pltpu.VMEM((4 * D,), jnp.float32),
      pltpu.VMEM((16,), jnp.float32),
      pltpu.VMEM((16,), jnp.int32),
      pltpu.SemaphoreType.DMA,
      pltpu.SemaphoreType.DMA,
      pltpu.SemaphoreType.DMA,
      pltpu.SemaphoreType.DMA,
  ]

  @functools.partial(pl.kernel, out_type=out_type, mesh=mesh,
                     scratch_types=scratch)
  def sc_reduce(ne, bt, ism, icnt, esid, ecnt, esm,
                rb0, rb1, bb, acc, cloc, esl, ec, eid,
                smr0, smr1, smb, smf):
    cid = lax.axis_index("c")
    sid = lax.axis_index("s")
    wid = sid * NC + cid
    row0 = wid * RPW

    def start_rows(c, rb, smr):
      pltpu.make_async_copy(ne.at[pl.ds(row0 + c * CHUNK, CHUNK)], rb,
                            smr).start()

    def wait_rows(rb, smr):
      pltpu.make_async_copy(ne.at[pl.ds(0, CHUNK)], rb, smr).wait()

    def wait_flush():
      pltpu.make_async_copy(acc.at[0], ism.at[0], smf).wait()

    pltpu.make_async_copy(bt.at[pl.ds(row0, RPW)],
                          bb.at[pl.ds(0, RPW)], smb).start()
    start_rows(0, rb0, smr0)

    def zcnt(i, _):
      cloc[pl.ds(i * 16, 16)] = jnp.zeros((16,), jnp.float32)
      return 0
    lax.fori_loop(0, G // 16, zcnt, 0)
    eid[...] = jnp.full((16,), -1, jnp.int32)
    ec[...] = jnp.zeros((16,), jnp.float32)

    pltpu.make_async_copy(bt.at[pl.ds(0, RPW)],
                          bb.at[pl.ds(0, RPW)], smb).wait()

    def save_edge(slot, seg, cnt, nfl):
      p = nfl & 1
      for j in range(NL):
        esl[pl.ds(slot * 2 * D + j * 16, 16)] = acc[p, pl.ds(j * 16, 16)]
        esl[pl.ds(slot * 2 * D + D + j * 16, 16)] = (
            acc[p, pl.ds(D + j * 16, 16)])
      _scalar_store(ec, slot, cnt.astype(jnp.float32))
      _scalar_store(eid, slot, seg)

    def make_row_body(rb):
      def row_body(args):
        i, ci, carry = args
        seg, cnt, nfl = carry
        b = bb[pl.ds(i, 16)][0]
        boundary = b != seg
        do_flush = jnp.logical_and(boundary, cnt > 0)

        @pl.when(do_flush)
        def _():
          p = nfl & 1

          @pl.when(nfl == 0)
          def _():
            save_edge(0, seg, cnt, nfl)

          @pl.when(nfl > 0)
          def _():
            @pl.when(nfl > 1)
            def _():
              wait_flush()
            pltpu.make_async_copy(acc.at[p], ism.at[seg], smf).start()
            _scalar_store(cloc, seg, cnt.astype(jnp.float32))

        newseg = jnp.where(boundary, b, seg)
        newcnt = jnp.where(boundary, 1, cnt + 1)
        newnfl = jnp.where(do_flush, nfl + 1, nfl)
        q = newnfl & 1
        for j in range(NL):
          v = rb[ci, pl.ds(j * 16, 16)]
          os_ = acc[q, pl.ds(j * 16, 16)]
          om_ = acc[q, pl.ds(D + j * 16, 16)]
          acc[q, pl.ds(j * 16, 16)] = jnp.where(boundary, v, os_ + v)
          acc[q, pl.ds(D + j * 16, 16)] = jnp.where(
              boundary, v, jnp.maximum(om_, v))
        return (newseg, newcnt, newnfl)
      return row_body

    def make_chunk_proc(rb):
      row_body = make_row_body(rb)

      def make_block(cbase, koff, W):
        def block_body(k, cr):
          seg = cr[0]
          base = koff + k * W
          first = bb[pl.ds(cbase + base, 16)][0]
          last = bb[pl.ds(cbase + base + W - 16, 16)][15]
          fast = jnp.logical_and(first == seg, last == seg)

          def fast_fn(c2):
            p = c2[2] & 1
            s_ = [acc[p, pl.ds(j * 16, 16)] for j in range(NL)]
            m_ = [acc[p, pl.ds(D + j * 16, 16)] for j in range(NL)]
            for r in range(W):
              for j in range(NL):
                v = rb[base + r, pl.ds(j * 16, 16)]
                s_[j] = s_[j] + v
                m_[j] = jnp.maximum(m_[j], v)
            for j in range(NL):
              acc[p, pl.ds(j * 16, 16)] = s_[j]
              acc[p, pl.ds(D + j * 16, 16)] = m_[j]
            return (c2[0], c2[1] + W, c2[2])

          def slow_fn(c2):
            def rb_body(r, c3):
              return row_body((cbase + base + r, base + r, c3))
            return lax.fori_loop(0, W, rb_body, c2)

          return lax.cond(fast, fast_fn, slow_fn, cr)
        return block_body

      def proc(carry, cbase):
        return lax.fori_loop(0, NBLK, make_block(cbase, 0, 16), carry)
      return proc

    proc0 = make_chunk_proc(rb0)
    proc1 = make_chunk_proc(rb1)

    def pair_body(i, carry):
      c0 = 2 * i
      start_rows(c0 + 1, rb1, smr1)
      wait_rows(rb0, smr0)
      carry = proc0(carry, c0 * CHUNK)
      start_rows(c0 + 2, rb0, smr0)
      wait_rows(rb1, smr1)
      return proc1(carry, (c0 + 1) * CHUNK)

    init = (jnp.int32(-1), jnp.int32(0), jnp.int32(0))
    carry = lax.fori_loop(0, (NCHUNK - 1) // 2, pair_body, init)
    wait_rows(rb0, smr0)
    carry = proc0(carry, (NCHUNK - 1) * CHUNK)

    seg, cnt, nfl = carry

    @pl.when(nfl == 0)
    def _():
      save_edge(0, seg, cnt, nfl)

    @pl.when(nfl > 0)
    def _():
      save_edge(1, seg, cnt, nfl)

    @pl.when(nfl >= 2)
    def _():
      wait_flush()

    pltpu.sync_copy(cloc, icnt.at[wid])
    pltpu.sync_copy(eid, esid.at[wid])
    pltpu.sync_copy(ec, ecnt.at[wid])
    pltpu.sync_copy(esl, esm.at[wid])

  return sc_reduce


_sc_reduce = _sc_reduce_build()


def _tc_head(ism, icnt, esid, esidv, ecntv, esm,
             ln_g, ln_b, W1, b1, W2, b2, Ws, bs, Wc, bc,
             score_out, cls_out, mx):
  dn = (((0,), (1,)), ((), ()))
  ones_w = jnp.ones((1, NW), jnp.float32)
  cnt_int = lax.dot_general(icnt[...], ones_w, dn,
                            preferred_element_type=jnp.float32)
  valid = cnt_int > 0.0

  isum = ism[:, pl.ds(0, D)]
  imax = ism[:, pl.ds(D, D)]

  mx[pl.ds(0, G), :] = jnp.where(valid, imax, NEG)
  mx[pl.ds(G, 8), :] = jnp.full((8, D), NEG, jnp.float32)

  def merge_max(w, _):
    for slot in range(2):
      sd = esid[w, slot]
      tid = jnp.where(sd >= 0, sd, G)
      rowf = esm[pl.ds(w, 1), :]
      row = lax.slice(rowf, (0, (2 * slot + 1) * D), (1, (2 * slot + 2) * D))
      cur = mx[pl.ds(tid, 1), :]
      mx[pl.ds(tid, 1), :] = jnp.maximum(cur, row)
    return 0
  lax.fori_loop(0, NW, merge_max, 0)

  ids = jax.lax.broadcasted_iota(jnp.int32, (NW, G), 1)
  oh0 = (esidv[:, pl.ds(0, 1)] == ids).astype(jnp.float32)
  oh1 = (esidv[:, pl.ds(1, 1)] == ids).astype(jnp.float32)
  dn0 = (((0,), (0,)), ((), ()))
  s = jnp.where(valid, isum, 0.0)
  s = s + lax.dot_general(oh0, esm[:, pl.ds(0, D)], dn0,
                          preferred_element_type=jnp.float32)
  s = s + lax.dot_general(oh1, esm[:, pl.ds(2 * D, D)], dn0,
                          preferred_element_type=jnp.float32)
  cnt = cnt_int
  cnt = cnt + lax.dot_general(oh0, ecntv[:, pl.ds(0, 1)], dn0,
                              preferred_element_type=jnp.float32)
  cnt = cnt + lax.dot_general(oh1, ecntv[:, pl.ds(1, 1)], dn0,
                              preferred_element_type=jnp.float32)

  mean = s / jnp.maximum(cnt, 1.0)
  mfin = jnp.where(cnt > 0.0, mx[pl.ds(0, G), :], 0.0)

  g = jnp.concatenate([s, mean, mfin], axis=1)
  mu = jnp.mean(g, axis=1, keepdims=True)
  var = jnp.mean((g - mu) ** 2, axis=1, keepdims=True)
  h = (g - mu) * jax.lax.rsqrt(var + EPS) * ln_g[...] + ln_b[...]

  h = h @ W1[...] + b1[...]
  h = h * jax.nn.sigmoid(h)
  h = h @ W2[...] + b2[...]
  h = h * jax.nn.sigmoid(h)
  score_out[...] = jax.nn.sigmoid(h @ Ws[...] + bs[...])
  cls_out[...] = h @ Wc[...] + bc[...]


@jax.jit
def kernel(node_emb, batch, ln_g, ln_b, W1, b1, W2, b2, Ws, bs, Wc, bc):
  ism, icnt, esid, ecnt, esm = _sc_reduce(node_emb, batch)

  vspec = pl.BlockSpec(memory_space=pltpu.VMEM)
  sspec = pl.BlockSpec(memory_space=pltpu.SMEM)
  score, cls = pl.pallas_call(
      _tc_head,
      out_shape=[jax.ShapeDtypeStruct((G, 1), jnp.float32),
                 jax.ShapeDtypeStruct((G, 4), jnp.float32)],
      in_specs=[vspec, vspec, sspec] + [vspec] * 13,
      out_specs=[vspec, vspec],
      scratch_shapes=[pltpu.VMEM((G + 8, D), jnp.float32)],
  )(ism, icnt, esid, esid, ecnt, esm,
    ln_g.reshape(1, 3 * D), ln_b.reshape(1, 3 * D), W1, b1.reshape(1, D),
    W2, b2.reshape(1, D // 2), Ws, bs.reshape(1, 1), Wc, bc.reshape(1, 4))
  return score[:, 0], cls
